# baseline (speedup 1.0000x reference)
import functools

import jax
import jax.numpy as jnp
from jax import lax
from jax.experimental import pallas as pl
from jax.experimental.pallas import tpu as pltpu
from jax.experimental.pallas import tpu_sc as plsc

D = 128
F = 64
LEVELS = 3
STEPS = 4

NC = 2
NS = 16
NW = NC * NS
CHUNK = 16
NBUF = 8
GRP = 40
DEG_CHUNK = 128
N_PAD = 10240
ROWS_PER_SUB = N_PAD // NS
DEG_W = 128
INV_W = 8
BLK = 512

_MESH = plsc.VectorSubcoreMesh(
    core_axis_name="c", subcore_axis_name="s", num_cores=NC, num_subcores=NS
)


def _make_edge_kernel(e_pad: int):
    chunks_per_tile = e_pad // (NW * CHUNK)
    ngroups = chunks_per_tile // GRP

    def body(h_hbm, src_hbm, dst_hbm, p_out, srcg, dstg, *scr):
        bufs = list(scr[:NBUF])
        acc = scr[NBUF]
        sems = list(scr[NBUF + 1:])
        b0 = bufs[0]
        cc = lax.axis_index("c")
        ss = lax.axis_index("s")
        wid = ss * NC + cc
        zeros16 = jnp.zeros((16,), jnp.float32)

        @pl.loop(0, CHUNK)
        def _(i):
            for j in range(D // 16):
                b0[i, pl.ds(j * 16, 16)] = zeros16

        rows0 = ss * ROWS_PER_SUB
        for k in range(ROWS_PER_SUB // CHUNK):
            pltpu.sync_copy(b0, acc.at[pl.ds(rows0 + k * CHUNK, CHUNK)])
        plsc.subcore_barrier()

        def gstart(r, b):
            pltpu.async_copy(h_hbm.at[srcg.at[r]], bufs[b], sems[b])

        def gwait(r, b):
            pltpu.make_async_copy(h_hbm.at[srcg.at[r]], bufs[b], sems[b]).wait()

        def scat(r, b):
            pltpu.sync_copy(bufs[b], acc.at[dstg.at[r]], add=True)

        tile_row0 = wid * chunks_per_tile

        @pl.loop(0, ngroups)
        def _(g):
            row0 = tile_row0 + g * GRP
            pltpu.sync_copy(src_hbm.at[pl.ds(row0, GRP)], srcg)
            pltpu.sync_copy(dst_hbm.at[pl.ds(row0, GRP)], dstg)
            for b in range(NBUF):
                gstart(b, b)

            @pl.loop(0, GRP // NBUF)
            def _(p):
                for b in range(NBUF):
                    r = p * NBUF + b
                    gwait(r, b)
                    scat(r, b)

                    @pl.when(r + NBUF < GRP)
                    def _():
                        gstart(r + NBUF, b)

        plsc.subcore_barrier()
        pltpu.sync_copy(
            acc.at[pl.ds(rows0, ROWS_PER_SUB)],
            p_out.at[cc, pl.ds(rows0, ROWS_PER_SUB)],
        )

    return pl.kernel(
        body,
        out_type=jax.ShapeDtypeStruct((NC, N_PAD, D), jnp.float32),
        mesh=_MESH,
        scratch_types=(
            [
                pltpu.VMEM((GRP, CHUNK), jnp.int32),
                pltpu.VMEM((GRP, CHUNK), jnp.int32),
            ]
            + [pltpu.VMEM((CHUNK, D), jnp.float32) for _ in range(NBUF)]
            + [pltpu.VMEM_SHARED((N_PAD, D), jnp.float32)]
            + [pltpu.SemaphoreType.DMA for _ in range(NBUF)]
        ),
    )


def _make_deg_kernel(e_pad: int):
    chunks_per_tile = e_pad // (NW * DEG_CHUNK)
    per_tile = chunks_per_tile * DEG_CHUNK

    def body(dst_hbm, pd_out, dst_v, ones_v, accd):
        cc = lax.axis_index("c")
        ss = lax.axis_index("s")
        wid = ss * NC + cc
        zeros16 = jnp.zeros((16,), jnp.float32)
        ones16 = jnp.ones((16,), jnp.float32)

        @pl.loop(0, DEG_CHUNK)
        def _(i):
            for j in range(DEG_W // 16):
                ones_v[i, pl.ds(j * 16, 16)] = zeros16

        rows0 = ss * ROWS_PER_SUB
        for k in range(ROWS_PER_SUB // DEG_CHUNK):
            pltpu.sync_copy(ones_v, accd.at[pl.ds(rows0 + k * DEG_CHUNK, DEG_CHUNK)])

        @pl.loop(0, DEG_CHUNK)
        def _(i):
            for j in range(DEG_W // 16):
                ones_v[i, pl.ds(j * 16, 16)] = ones16

        plsc.subcore_barrier()

        base = wid * per_tile

        @pl.loop(0, chunks_per_tile)
        def _(ci):
            off = base + ci * DEG_CHUNK
            pltpu.sync_copy(dst_hbm.at[pl.ds(off, DEG_CHUNK)], dst_v)
            pltpu.sync_copy(ones_v, accd.at[dst_v], add=True)

        plsc.subcore_barrier()
        pltpu.sync_copy(
            accd.at[pl.ds(rows0, ROWS_PER_SUB)],
            pd_out.at[cc, pl.ds(rows0, ROWS_PER_SUB)],
        )

    return pl.kernel(
        body,
        out_type=jax.ShapeDtypeStruct((NC, N_PAD, DEG_W), jnp.float32),
        mesh=_MESH,
        scratch_types=[
            pltpu.VMEM((DEG_CHUNK,), jnp.int32),
            pltpu.VMEM((DEG_CHUNK, DEG_W), jnp.float32),
            pltpu.VMEM_SHARED((N_PAD, DEG_W), jnp.float32),
        ],
    )


def _inv_deg_body(pd_ref, inv_ref):
    pd = pd_ref[...]
    deg = pd[0, :, :1] + pd[1, :, :1] + 1.0
    inv_ref[...] = jnp.broadcast_to(1.0 / deg, (deg.shape[0], INV_W))


_inv_deg = pl.pallas_call(
    _inv_deg_body,
    grid=(N_PAD // BLK,),
    in_specs=[pl.BlockSpec((NC, BLK, DEG_W), lambda i: (0, i, 0))],
    out_specs=pl.BlockSpec((BLK, INV_W), lambda i: (i, 0)),
    out_shape=jax.ShapeDtypeStruct((N_PAD, INV_W), jnp.float32),
)


def _combine_body(inv_ref, hp_ref, p_ref, h_ref):
    inv = inv_ref[...][:, :1]
    p = p_ref[...]
    h_ref[...] = (p[0] + p[1] + hp_ref[...]) * inv


_combine = pl.pallas_call(
    _combine_body,
    grid=(N_PAD // BLK,),
    in_specs=[
        pl.BlockSpec((BLK, INV_W), lambda i: (i, 0)),
        pl.BlockSpec((BLK, D), lambda i: (i, 0)),
        pl.BlockSpec((NC, BLK, D), lambda i: (0, i, 0)),
    ],
    out_specs=pl.BlockSpec((BLK, D), lambda i: (i, 0)),
    out_shape=jax.ShapeDtypeStruct((N_PAD, D), jnp.float32),
)


def _ls2t_body(h0, h1, h2, h3, p4_ref, inv_ref, k_ref, b_ref, e_ref, o_ref):
    p4 = p4_ref[...]
    h3v = h3[...]
    h4 = (p4[0] + p4[1] + h3v) * inv_ref[...][:, :1]
    hs = [h0[...], h1[...], h2[...], h3v, h4]
    diffs = [hs[0]] + [hs[l + 1] - hs[l] for l in range(STEPS)]
    K = k_ref[...]
    b = b_ref[...]
    M = [
        [
            jnp.dot(diffs[l], K[m], preferred_element_type=jnp.float32) + b[m]
            for l in range(STEPS + 1)
        ]
        for m in range(LEVELS)
    ]
    Y = M[0]
    res = [sum(Y[1:], Y[0])]
    for m in range(1, LEVELS):
        c = jnp.zeros_like(Y[0])
        newY = []
        for l in range(STEPS + 1):
            newY.append(M[m][l] * c)
            c = c + Y[l]
        Y = newY
        res.append(sum(Y[1:], Y[0]))
    out = jnp.stack(res, axis=1) * e_ref[...][None]
    o_ref[...] = out


@functools.cache
def _make_ls2t(n: int):
    return pl.pallas_call(
        _ls2t_body,
        grid=(N_PAD // BLK,),
        in_specs=[pl.BlockSpec((BLK, D), lambda i: (i, 0)) for _ in range(STEPS)]
        + [
            pl.BlockSpec((NC, BLK, D), lambda i: (0, i, 0)),
            pl.BlockSpec((BLK, INV_W), lambda i: (i, 0)),
        ]
        + [
            pl.BlockSpec((LEVELS, D, F), lambda i: (0, 0, 0)),
            pl.BlockSpec((LEVELS, F), lambda i: (0, 0)),
            pl.BlockSpec((LEVELS, F), lambda i: (0, 0)),
        ],
        out_specs=pl.BlockSpec((BLK, LEVELS, F), lambda i: (i, 0, 0)),
        out_shape=jax.ShapeDtypeStruct((n, LEVELS, F), jnp.float32),
    )


def kernel(x, edge_index, kernel, bias, embed_coeffs):
    n = x.shape[0]
    dst = edge_index[:, 0]
    src = edge_index[:, 1]
    e = dst.shape[0]
    gran = NW * CHUNK * GRP
    e_pad = ((e + gran - 1) // gran) * gran
    pad_idx = n + (jnp.arange(e_pad - e, dtype=jnp.int32) % (N_PAD - n))
    dst_p = jnp.concatenate([dst, pad_idx])
    src_p = jnp.concatenate([src, pad_idx])
    src2 = src_p.reshape(-1, CHUNK)
    dst2 = dst_p.reshape(-1, CHUNK)
    x_p = jnp.zeros((N_PAD, D), jnp.float32).at[:n].set(x)

    step = _make_edge_kernel(e_pad)
    deg = _make_deg_kernel(e_pad)

    pd = deg(dst_p)
    inv = _inv_deg(pd)
    hs = [x_p]
    h = x_p
    for _ in range(STEPS - 1):
        p = step(h, src2, dst2)
        h = _combine(inv, h, p)
        hs.append(h)
    p4 = step(h, src2, dst2)

    ls2t = _make_ls2t(n)
    return ls2t(hs[0], hs[1], hs[2], hs[3], p4, inv, kernel, bias, embed_coeffs)


# speedup vs baseline: 1.1438x; 1.1438x over previous
import functools

import jax
import jax.numpy as jnp
from jax import lax
from jax.experimental import pallas as pl
from jax.experimental.pallas import tpu as pltpu
from jax.experimental.pallas import tpu_sc as plsc

D = 128
F = 64
LEVELS = 3
STEPS = 4

NC = 2
NS = 16
NW = NC * NS
CHUNK = 32
NBUF = 4
GRP = 40
DEG_CHUNK = 128
N_PAD = 10240
ROWS_PER_SUB = N_PAD // NS
DEG_W = 128
INV_W = 8
BLK = 512

_MESH = plsc.VectorSubcoreMesh(
    core_axis_name="c", subcore_axis_name="s", num_cores=NC, num_subcores=NS
)


def _make_edge_kernel(e_pad: int):
    chunks_per_tile = e_pad // (NW * CHUNK)
    ngroups = chunks_per_tile // GRP

    def body(h_hbm, src_hbm, dst_hbm, p_out, srcg, dstg, *scr):
        bufs = list(scr[:NBUF])
        acc = scr[NBUF]
        sems = list(scr[NBUF + 1:])
        b0 = bufs[0]
        cc = lax.axis_index("c")
        ss = lax.axis_index("s")
        wid = ss * NC + cc
        zeros16 = jnp.zeros((16,), jnp.float32)

        @pl.loop(0, CHUNK)
        def _(i):
            for j in range(D // 16):
                b0[i, pl.ds(j * 16, 16)] = zeros16

        rows0 = ss * ROWS_PER_SUB
        for k in range(ROWS_PER_SUB // CHUNK):
            pltpu.sync_copy(b0, acc.at[pl.ds(rows0 + k * CHUNK, CHUNK)])
        plsc.subcore_barrier()

        def gstart(r, b):
            pltpu.async_copy(h_hbm.at[srcg.at[r]], bufs[b], sems[b])

        def gwait(r, b):
            pltpu.make_async_copy(h_hbm.at[srcg.at[r]], bufs[b], sems[b]).wait()

        def scat(r, b):
            pltpu.sync_copy(bufs[b], acc.at[dstg.at[r]], add=True)

        tile_row0 = wid * chunks_per_tile

        @pl.loop(0, ngroups)
        def _(g):
            row0 = tile_row0 + g * GRP
            pltpu.sync_copy(src_hbm.at[pl.ds(row0, GRP)], srcg)
            pltpu.sync_copy(dst_hbm.at[pl.ds(row0, GRP)], dstg)
            for b in range(NBUF):
                gstart(b, b)

            @pl.loop(0, GRP // NBUF)
            def _(p):
                for b in range(NBUF):
                    r = p * NBUF + b
                    gwait(r, b)
                    scat(r, b)

                    @pl.when(r + NBUF < GRP)
                    def _():
                        gstart(r + NBUF, b)

        plsc.subcore_barrier()
        pltpu.sync_copy(
            acc.at[pl.ds(rows0, ROWS_PER_SUB)],
            p_out.at[cc, pl.ds(rows0, ROWS_PER_SUB)],
        )

    return pl.kernel(
        body,
        out_type=jax.ShapeDtypeStruct((NC, N_PAD, D), jnp.float32),
        mesh=_MESH,
        scratch_types=(
            [
                pltpu.VMEM((GRP, CHUNK), jnp.int32),
                pltpu.VMEM((GRP, CHUNK), jnp.int32),
            ]
            + [pltpu.VMEM((CHUNK, D), jnp.float32) for _ in range(NBUF)]
            + [pltpu.VMEM_SHARED((N_PAD, D), jnp.float32)]
            + [pltpu.SemaphoreType.DMA for _ in range(NBUF)]
        ),
    )


def _make_deg_kernel(e_pad: int):
    chunks_per_tile = e_pad // (NW * DEG_CHUNK)
    per_tile = chunks_per_tile * DEG_CHUNK

    def body(dst_hbm, pd_out, dst_v, ones_v, accd):
        cc = lax.axis_index("c")
        ss = lax.axis_index("s")
        wid = ss * NC + cc
        zeros16 = jnp.zeros((16,), jnp.float32)
        ones16 = jnp.ones((16,), jnp.float32)

        @pl.loop(0, DEG_CHUNK)
        def _(i):
            for j in range(DEG_W // 16):
                ones_v[i, pl.ds(j * 16, 16)] = zeros16

        rows0 = ss * ROWS_PER_SUB
        for k in range(ROWS_PER_SUB // DEG_CHUNK):
            pltpu.sync_copy(ones_v, accd.at[pl.ds(rows0 + k * DEG_CHUNK, DEG_CHUNK)])

        @pl.loop(0, DEG_CHUNK)
        def _(i):
            for j in range(DEG_W // 16):
                ones_v[i, pl.ds(j * 16, 16)] = ones16

        plsc.subcore_barrier()

        base = wid * per_tile

        @pl.loop(0, chunks_per_tile)
        def _(ci):
            off = base + ci * DEG_CHUNK
            pltpu.sync_copy(dst_hbm.at[pl.ds(off, DEG_CHUNK)], dst_v)
            pltpu.sync_copy(ones_v, accd.at[dst_v], add=True)

        plsc.subcore_barrier()
        pltpu.sync_copy(
            accd.at[pl.ds(rows0, ROWS_PER_SUB)],
            pd_out.at[cc, pl.ds(rows0, ROWS_PER_SUB)],
        )

    return pl.kernel(
        body,
        out_type=jax.ShapeDtypeStruct((NC, N_PAD, DEG_W), jnp.float32),
        mesh=_MESH,
        scratch_types=[
            pltpu.VMEM((DEG_CHUNK,), jnp.int32),
            pltpu.VMEM((DEG_CHUNK, DEG_W), jnp.float32),
            pltpu.VMEM_SHARED((N_PAD, DEG_W), jnp.float32),
        ],
    )


def _inv_deg_body(pd_ref, inv_ref):
    pd = pd_ref[...]
    deg = pd[0, :, :1] + pd[1, :, :1] + 1.0
    inv_ref[...] = jnp.broadcast_to(1.0 / deg, (deg.shape[0], INV_W))


_inv_deg = pl.pallas_call(
    _inv_deg_body,
    grid=(N_PAD // BLK,),
    in_specs=[pl.BlockSpec((NC, BLK, DEG_W), lambda i: (0, i, 0))],
    out_specs=pl.BlockSpec((BLK, INV_W), lambda i: (i, 0)),
    out_shape=jax.ShapeDtypeStruct((N_PAD, INV_W), jnp.float32),
)


def _combine_body(inv_ref, hp_ref, p_ref, h_ref):
    inv = inv_ref[...][:, :1]
    p = p_ref[...]
    h_ref[...] = (p[0] + p[1] + hp_ref[...]) * inv


_combine = pl.pallas_call(
    _combine_body,
    grid=(N_PAD // BLK,),
    in_specs=[
        pl.BlockSpec((BLK, INV_W), lambda i: (i, 0)),
        pl.BlockSpec((BLK, D), lambda i: (i, 0)),
        pl.BlockSpec((NC, BLK, D), lambda i: (0, i, 0)),
    ],
    out_specs=pl.BlockSpec((BLK, D), lambda i: (i, 0)),
    out_shape=jax.ShapeDtypeStruct((N_PAD, D), jnp.float32),
)


def _ls2t_body(h0, h1, h2, h3, p4_ref, inv_ref, k_ref, b_ref, e_ref, o_ref):
    p4 = p4_ref[...]
    h3v = h3[...]
    h4 = (p4[0] + p4[1] + h3v) * inv_ref[...][:, :1]
    hs = [h0[...], h1[...], h2[...], h3v, h4]
    diffs = [hs[0]] + [hs[l + 1] - hs[l] for l in range(STEPS)]
    K = k_ref[...]
    b = b_ref[...]
    M = [
        [
            jnp.dot(diffs[l], K[m], preferred_element_type=jnp.float32) + b[m]
            for l in range(STEPS + 1)
        ]
        for m in range(LEVELS)
    ]
    Y = M[0]
    res = [sum(Y[1:], Y[0])]
    for m in range(1, LEVELS):
        c = jnp.zeros_like(Y[0])
        newY = []
        for l in range(STEPS + 1):
            newY.append(M[m][l] * c)
            c = c + Y[l]
        Y = newY
        res.append(sum(Y[1:], Y[0]))
    out = jnp.stack(res, axis=1) * e_ref[...][None]
    o_ref[...] = out


@functools.cache
def _make_ls2t(n: int):
    return pl.pallas_call(
        _ls2t_body,
        grid=(N_PAD // BLK,),
        in_specs=[pl.BlockSpec((BLK, D), lambda i: (i, 0)) for _ in range(STEPS)]
        + [
            pl.BlockSpec((NC, BLK, D), lambda i: (0, i, 0)),
            pl.BlockSpec((BLK, INV_W), lambda i: (i, 0)),
        ]
        + [
            pl.BlockSpec((LEVELS, D, F), lambda i: (0, 0, 0)),
            pl.BlockSpec((LEVELS, F), lambda i: (0, 0)),
            pl.BlockSpec((LEVELS, F), lambda i: (0, 0)),
        ],
        out_specs=pl.BlockSpec((BLK, LEVELS, F), lambda i: (i, 0, 0)),
        out_shape=jax.ShapeDtypeStruct((n, LEVELS, F), jnp.float32),
    )


def kernel(x, edge_index, kernel, bias, embed_coeffs):
    n = x.shape[0]
    dst = edge_index[:, 0]
    src = edge_index[:, 1]
    e = dst.shape[0]
    gran = NW * CHUNK * GRP
    e_pad = ((e + gran - 1) // gran) * gran
    pad_idx = n + (jnp.arange(e_pad - e, dtype=jnp.int32) % (N_PAD - n))
    dst_p = jnp.concatenate([dst, pad_idx])
    src_p = jnp.concatenate([src, pad_idx])
    src2 = src_p.reshape(-1, CHUNK)
    dst2 = dst_p.reshape(-1, CHUNK)
    x_p = jnp.zeros((N_PAD, D), jnp.float32).at[:n].set(x)

    step = _make_edge_kernel(e_pad)
    deg = _make_deg_kernel(e_pad)

    pd = deg(dst_p)
    inv = _inv_deg(pd)
    hs = [x_p]
    h = x_p
    for _ in range(STEPS - 1):
        p = step(h, src2, dst2)
        h = _combine(inv, h, p)
        hs.append(h)
    p4 = step(h, src2, dst2)

    ls2t = _make_ls2t(n)
    return ls2t(hs[0], hs[1], hs[2], hs[3], p4, inv, kernel, bias, embed_coeffs)


# speedup vs baseline: 1.1712x; 1.0240x over previous
import functools

import jax
import jax.numpy as jnp
from jax import lax
from jax.experimental import pallas as pl
from jax.experimental.pallas import tpu as pltpu
from jax.experimental.pallas import tpu_sc as plsc

D = 128
F = 64
LEVELS = 3
STEPS = 4

NC = 2
NS = 16
NW = NC * NS
CHUNK = 32
NBUF = 4
GRP = 40
DEG_CHUNK = 128
N_PAD = 10240
ROWS_PER_SUB = N_PAD // NS
DEG_W = 128
INV_W = 8
BLK = 512

_MESH = plsc.VectorSubcoreMesh(
    core_axis_name="c", subcore_axis_name="s", num_cores=NC, num_subcores=NS
)


def _make_edge_kernel(e_pad: int):
    chunks_per_tile = e_pad // (NW * CHUNK)
    ngroups = chunks_per_tile // GRP

    def body(h_hbm, src_hbm, dst_hbm, p_out, srcg, dstg, *scr):
        bufs = list(scr[:NBUF])
        acc = scr[NBUF]
        sems = list(scr[NBUF + 1:])
        b0 = bufs[0]
        cc = lax.axis_index("c")
        ss = lax.axis_index("s")
        wid = ss * NC + cc
        zeros16 = jnp.zeros((16,), jnp.float32)

        def gstart(r, b):
            pltpu.async_copy(h_hbm.at[srcg.at[r]], bufs[b], sems[b])

        def gwait(r, b):
            pltpu.make_async_copy(h_hbm.at[srcg.at[r]], bufs[b], sems[b]).wait()

        def scat(r, b):
            pltpu.sync_copy(bufs[b], acc.at[dstg.at[r]], add=True)

        tile_row0 = wid * chunks_per_tile

        pltpu.sync_copy(src_hbm.at[pl.ds(tile_row0, GRP)], srcg)
        pltpu.sync_copy(dst_hbm.at[pl.ds(tile_row0, GRP)], dstg)
        for b in range(1, NBUF):
            gstart(b, b)

        @pl.loop(0, CHUNK)
        def _(i):
            for j in range(D // 16):
                b0[i, pl.ds(j * 16, 16)] = zeros16

        rows0 = ss * ROWS_PER_SUB
        for k in range(ROWS_PER_SUB // CHUNK):
            pltpu.sync_copy(b0, acc.at[pl.ds(rows0 + k * CHUNK, CHUNK)])
        gstart(0, 0)
        plsc.subcore_barrier()

        @pl.loop(0, ngroups)
        def _(g):
            row0 = tile_row0 + g * GRP

            @pl.when(g > 0)
            def _():
                pltpu.sync_copy(src_hbm.at[pl.ds(row0, GRP)], srcg)
                pltpu.sync_copy(dst_hbm.at[pl.ds(row0, GRP)], dstg)
                for b in range(NBUF):
                    gstart(b, b)

            @pl.loop(0, GRP // NBUF)
            def _(p):
                for b in range(NBUF):
                    r = p * NBUF + b
                    gwait(r, b)
                    scat(r, b)

                    @pl.when(r + NBUF < GRP)
                    def _():
                        gstart(r + NBUF, b)

        plsc.subcore_barrier()
        pltpu.sync_copy(
            acc.at[pl.ds(rows0, ROWS_PER_SUB)],
            p_out.at[cc, pl.ds(rows0, ROWS_PER_SUB)],
        )

    return pl.kernel(
        body,
        out_type=jax.ShapeDtypeStruct((NC, N_PAD, D), jnp.float32),
        mesh=_MESH,
        scratch_types=(
            [
                pltpu.VMEM((GRP, CHUNK), jnp.int32),
                pltpu.VMEM((GRP, CHUNK), jnp.int32),
            ]
            + [pltpu.VMEM((CHUNK, D), jnp.float32) for _ in range(NBUF)]
            + [pltpu.VMEM_SHARED((N_PAD, D), jnp.float32)]
            + [pltpu.SemaphoreType.DMA for _ in range(NBUF)]
        ),
    )


def _make_deg_kernel(e_pad: int):
    chunks_per_tile = e_pad // (NW * DEG_CHUNK)
    per_tile = chunks_per_tile * DEG_CHUNK

    def body(dst_hbm, pd_out, dst_v, ones_v, accd):
        cc = lax.axis_index("c")
        ss = lax.axis_index("s")
        wid = ss * NC + cc
        zeros16 = jnp.zeros((16,), jnp.float32)
        ones16 = jnp.ones((16,), jnp.float32)

        @pl.loop(0, DEG_CHUNK)
        def _(i):
            for j in range(DEG_W // 16):
                ones_v[i, pl.ds(j * 16, 16)] = zeros16

        rows0 = ss * ROWS_PER_SUB
        for k in range(ROWS_PER_SUB // DEG_CHUNK):
            pltpu.sync_copy(ones_v, accd.at[pl.ds(rows0 + k * DEG_CHUNK, DEG_CHUNK)])

        @pl.loop(0, DEG_CHUNK)
        def _(i):
            for j in range(DEG_W // 16):
                ones_v[i, pl.ds(j * 16, 16)] = ones16

        plsc.subcore_barrier()

        base = wid * per_tile

        @pl.loop(0, chunks_per_tile)
        def _(ci):
            off = base + ci * DEG_CHUNK
            pltpu.sync_copy(dst_hbm.at[pl.ds(off, DEG_CHUNK)], dst_v)
            pltpu.sync_copy(ones_v, accd.at[dst_v], add=True)

        plsc.subcore_barrier()
        pltpu.sync_copy(
            accd.at[pl.ds(rows0, ROWS_PER_SUB)],
            pd_out.at[cc, pl.ds(rows0, ROWS_PER_SUB)],
        )

    return pl.kernel(
        body,
        out_type=jax.ShapeDtypeStruct((NC, N_PAD, DEG_W), jnp.float32),
        mesh=_MESH,
        scratch_types=[
            pltpu.VMEM((DEG_CHUNK,), jnp.int32),
            pltpu.VMEM((DEG_CHUNK, DEG_W), jnp.float32),
            pltpu.VMEM_SHARED((N_PAD, DEG_W), jnp.float32),
        ],
    )


def _inv_deg_body(pd_ref, inv_ref):
    pd = pd_ref[...]
    deg = pd[0, :, :1] + pd[1, :, :1] + 1.0
    inv_ref[...] = jnp.broadcast_to(1.0 / deg, (deg.shape[0], INV_W))


_inv_deg = pl.pallas_call(
    _inv_deg_body,
    grid=(N_PAD // BLK,),
    in_specs=[pl.BlockSpec((NC, BLK, DEG_W), lambda i: (0, i, 0))],
    out_specs=pl.BlockSpec((BLK, INV_W), lambda i: (i, 0)),
    out_shape=jax.ShapeDtypeStruct((N_PAD, INV_W), jnp.float32),
)


def _combine_body(inv_ref, hp_ref, p_ref, h_ref):
    inv = inv_ref[...][:, :1]
    p = p_ref[...]
    h_ref[...] = (p[0] + p[1] + hp_ref[...]) * inv


_combine = pl.pallas_call(
    _combine_body,
    grid=(N_PAD // BLK,),
    in_specs=[
        pl.BlockSpec((BLK, INV_W), lambda i: (i, 0)),
        pl.BlockSpec((BLK, D), lambda i: (i, 0)),
        pl.BlockSpec((NC, BLK, D), lambda i: (0, i, 0)),
    ],
    out_specs=pl.BlockSpec((BLK, D), lambda i: (i, 0)),
    out_shape=jax.ShapeDtypeStruct((N_PAD, D), jnp.float32),
)


def _ls2t_body(h0, h1, h2, h3, p4_ref, inv_ref, k_ref, b_ref, e_ref, o_ref):
    p4 = p4_ref[...]
    h3v = h3[...]
    h4 = (p4[0] + p4[1] + h3v) * inv_ref[...][:, :1]
    hs = [h0[...], h1[...], h2[...], h3v, h4]
    diffs = [hs[0]] + [hs[l + 1] - hs[l] for l in range(STEPS)]
    K = k_ref[...]
    b = b_ref[...]
    M = [
        [
            jnp.dot(diffs[l], K[m], preferred_element_type=jnp.float32) + b[m]
            for l in range(STEPS + 1)
        ]
        for m in range(LEVELS)
    ]
    Y = M[0]
    res = [sum(Y[1:], Y[0])]
    for m in range(1, LEVELS):
        c = jnp.zeros_like(Y[0])
        newY = []
        for l in range(STEPS + 1):
            newY.append(M[m][l] * c)
            c = c + Y[l]
        Y = newY
        res.append(sum(Y[1:], Y[0]))
    e = e_ref[...]
    o_ref[...] = jnp.concatenate([res[m] * e[m] for m in range(LEVELS)], axis=1)


@functools.cache
def _make_ls2t(n: int):
    return pl.pallas_call(
        _ls2t_body,
        grid=(N_PAD // BLK,),
        in_specs=[pl.BlockSpec((BLK, D), lambda i: (i, 0)) for _ in range(STEPS)]
        + [
            pl.BlockSpec((NC, BLK, D), lambda i: (0, i, 0)),
            pl.BlockSpec((BLK, INV_W), lambda i: (i, 0)),
        ]
        + [
            pl.BlockSpec((LEVELS, D, F), lambda i: (0, 0, 0)),
            pl.BlockSpec((LEVELS, F), lambda i: (0, 0)),
            pl.BlockSpec((LEVELS, F), lambda i: (0, 0)),
        ],
        out_specs=pl.BlockSpec((BLK, LEVELS * F), lambda i: (i, 0)),
        out_shape=jax.ShapeDtypeStruct((n, LEVELS * F), jnp.float32),
    )


def kernel(x, edge_index, kernel, bias, embed_coeffs):
    n = x.shape[0]
    dst = edge_index[:, 0]
    src = edge_index[:, 1]
    e = dst.shape[0]
    gran = NW * CHUNK * GRP
    e_pad = ((e + gran - 1) // gran) * gran
    pad_idx = n + (jnp.arange(e_pad - e, dtype=jnp.int32) % (N_PAD - n))
    dst_p = jnp.concatenate([dst, pad_idx])
    src_p = jnp.concatenate([src, pad_idx])
    src2 = src_p.reshape(-1, CHUNK)
    dst2 = dst_p.reshape(-1, CHUNK)
    x_p = jnp.zeros((N_PAD, D), jnp.float32).at[:n].set(x)

    step = _make_edge_kernel(e_pad)
    deg = _make_deg_kernel(e_pad)

    pd = deg(dst_p)
    inv = _inv_deg(pd)
    hs = [x_p]
    h = x_p
    for _ in range(STEPS - 1):
        p = step(h, src2, dst2)
        h = _combine(inv, h, p)
        hs.append(h)
    p4 = step(h, src2, dst2)

    ls2t = _make_ls2t(n)
    out = ls2t(hs[0], hs[1], hs[2], hs[3], p4, inv, kernel, bias, embed_coeffs)
    return out.reshape(n, LEVELS, F)
